# baseline (speedup 1.0000x reference)
import functools

import numpy as np

import jax
import jax.numpy as jnp
from jax import lax
from jax.experimental import pallas as pl
from jax.experimental.pallas import tpu as pltpu
from jax.experimental.pallas import tpu_sc as plsc

B, H, W, C = 2, 224, 224, 96
C2 = 2 * C
BC = B * C
HW = H * W

ROW_TILE = 8
NT = H // ROW_TILE
PIX = ROW_TILE * W
HPIX = PIX // 2
HALF = HW // 2

NW = 32
PLANES_PER_W = BC // NW
NCHUNKS = 28
CHUNK = HW // NCHUNKS
ROWS_PER_CHUNK = CHUNK // W
VECS_PER_ROW = W // 16


def _grid_consts():
    l = np.arange(PIX)
    even = (l % 2 == 0)
    g = np.zeros((2, PIX), np.float32)
    for half in range(2):
        g[half] = np.where(even, half * (HALF // W) + (l // 2) // W,
                           (l // 2) % W)
    m = np.broadcast_to(even.astype(np.float32), (2, PIX)).copy()
    return g, m


_G_BASE, _G_TMASK = _grid_consts()


def _conv_body(x_ref, w_ref, b_ref, g_ref, m_ref, co, xt):
    t = pl.program_id(1)
    r0 = t * ROW_TILE
    acc = jnp.broadcast_to(b_ref[0][:, None], (C2, PIX))
    for dy in range(3):
        rows = x_ref[0, pl.ds(r0 + dy, ROW_TILE), :, :]
        for dx in range(3):
            blk = rows[:, dx:dx + W, :].reshape(PIX, C)
            acc = acc + lax.dot_general(
                w_ref[dy, dx], blk, (((0,), (1,)), ((), ())),
                preferred_element_type=jnp.float32)
    g = g_ref[...] + (t * (ROW_TILE // 2)).astype(jnp.float32) * m_ref[...]
    coords = acc.reshape(C, 2, PIX) + g[None, :, :]
    coords = jnp.minimum(jnp.maximum(coords, 0.0), jnp.float32(H - 1))
    co[0, :, :] = coords.reshape(C2, PIX)
    xmid = x_ref[0, pl.ds(r0 + 1, ROW_TILE), pl.ds(1, W), :]
    xt[0, :, :] = xmid.reshape(PIX, C).T


def _conv_coords(x, W_conv, b_conv):
    x_pad = jnp.pad(x, ((0, 0), (1, 1), (1, 1), (0, 0)))
    return pl.pallas_call(
        _conv_body,
        grid=(B, NT),
        in_specs=[
            pl.BlockSpec((1, H + 2, W + 2, C), lambda bi, ti: (bi, 0, 0, 0)),
            pl.BlockSpec((3, 3, C, C2), lambda bi, ti: (0, 0, 0, 0)),
            pl.BlockSpec((1, C2), lambda bi, ti: (0, 0)),
            pl.BlockSpec((2, PIX), lambda bi, ti: (0, 0)),
            pl.BlockSpec((2, PIX), lambda bi, ti: (0, 0)),
        ],
        out_specs=[
            pl.BlockSpec((1, C2, PIX), lambda bi, ti: (bi, 0, ti)),
            pl.BlockSpec((1, C, PIX), lambda bi, ti: (bi, 0, ti)),
        ],
        out_shape=[
            jax.ShapeDtypeStruct((B, C2, HW), jnp.float32),
            jax.ShapeDtypeStruct((B, C, HW), jnp.float32),
        ],
        compiler_params=pltpu.CompilerParams(
            vmem_limit_bytes=100 * 1024 * 1024),
    )(x_pad, W_conv, b_conv.reshape(1, C2), _G_BASE, _G_TMASK)


def _transpose_body(i_ref, o_ref):
    o_ref[0] = i_ref[0].T.reshape(ROW_TILE, W, C)


def _to_nhwc(planes):
    return pl.pallas_call(
        _transpose_body,
        grid=(B, NT),
        in_specs=[pl.BlockSpec((1, C, PIX), lambda bi, ti: (bi, 0, ti))],
        out_specs=pl.BlockSpec((1, ROW_TILE, W, C),
                               lambda bi, ti: (bi, ti, 0, 0)),
        out_shape=jax.ShapeDtypeStruct((B, H, W, C), jnp.float32),
    )(planes)


@functools.lru_cache(maxsize=1)
def _build_sc_sample():
    mesh = plsc.VectorSubcoreMesh(core_axis_name="c", subcore_axis_name="s")
    return functools.partial(
        pl.kernel,
        mesh=mesh,
        out_type=jax.ShapeDtypeStruct((BC, HW), jnp.float32),
        compiler_params=pltpu.CompilerParams(needs_layout_passes=False),
        scratch_types=[
            pltpu.VMEM((2, HW), jnp.float32),
            pltpu.VMEM((3, 2 * CHUNK), jnp.float32),
            pltpu.VMEM((2, CHUNK), jnp.float32),
        ],
    )(_sc_sample_body)


def _sc_sample_body(xp_hbm, co_hbm, out_hbm, plane_v, c_v, res_v):
    wid = lax.axis_index("s") * 2 + lax.axis_index("c")
    iota = lax.iota(jnp.int32, 16)
    ev = iota * 2
    p0 = wid * PLANES_PER_W

    def fetch_coords(b, ci, k, slot):
        half = k // (NCHUNKS // 2)
        col0 = (k - half * (NCHUNKS // 2)) * (2 * CHUNK)
        pltpu.sync_copy(
            co_hbm.at[b, 2 * ci + half, pl.ds(col0, 2 * CHUNK)],
            c_v.at[slot])

    pltpu.sync_copy(xp_hbm.at[p0], plane_v.at[0])

    def plane_body(j, _):
        p = p0 + j
        pp = j % 2
        b = p // C
        ci = p - b * C
        ppv = jnp.full((16,), pp, jnp.int32)

        @pl.when(j < PLANES_PER_W - 1)
        def _():
            pltpu.sync_copy(xp_hbm.at[p + 1], plane_v.at[(j + 1) % 2])

        fetch_coords(b, ci, 0, 0)
        fetch_coords(b, ci, 1, 1)

        def chunk_body(k, _):
            @pl.when(k < NCHUNKS - 2)
            def _():
                fetch_coords(b, ci, k + 2, (k + 2) % 3)

            @pl.when(k > 0)
            def _():
                pltpu.sync_copy(
                    res_v.at[(k - 1) % 2],
                    out_hbm.at[p, pl.ds((k - 1) * CHUNK, CHUNK)])

            cslot = jnp.full((16,), k % 3, jnp.int32)
            rpar = k % 2

            def row_body(r, _):
                for v in range(VECS_PER_ROW):
                    q = r * W + v * 16
                    cidx = ev + (2 * q)
                    cy = plsc.load_gather(c_v, [cslot, cidx])
                    cx = plsc.load_gather(c_v, [cslot, cidx + 1])
                    iy0 = cy.astype(jnp.int32)
                    ix0 = cx.astype(jnp.int32)
                    wy = cy - iy0.astype(jnp.float32)
                    wx = cx - ix0.astype(jnp.float32)
                    ady = jnp.where(iy0 < H - 1, W, 0)
                    adx = jnp.where(ix0 < W - 1, 1, 0)
                    i_lt = iy0 * W + ix0
                    i_rt = i_lt + ady
                    v_lt = plsc.load_gather(plane_v, [ppv, i_lt])
                    v_rt = plsc.load_gather(plane_v, [ppv, i_rt])
                    v_lb = plsc.load_gather(plane_v, [ppv, i_lt + adx])
                    v_rb = plsc.load_gather(plane_v, [ppv, i_rt + adx])
                    vt = v_lt + (v_rt - v_lt) * wy
                    vb = v_lb + (v_rb - v_lb) * wy
                    res_v[rpar, pl.ds(q, 16)] = vt + (vb - vt) * wx
                return 0

            lax.fori_loop(0, ROWS_PER_CHUNK, row_body, 0, unroll=2)
            return 0

        lax.fori_loop(0, NCHUNKS, chunk_body, 0)
        pltpu.sync_copy(
            res_v.at[(NCHUNKS - 1) % 2],
            out_hbm.at[p, pl.ds((NCHUNKS - 1) * CHUNK, CHUNK)])
        return 0

    lax.fori_loop(0, PLANES_PER_W, plane_body, 0)


def kernel(x, W_conv, b_conv):
    co, xt = _conv_coords(x, W_conv, b_conv)
    x_bc = xt.reshape(BC, HW)
    planes = _build_sc_sample()(x_bc, co)
    return _to_nhwc(planes.reshape(B, C, HW))


# speedup vs baseline: 1.0502x; 1.0502x over previous
import functools

import numpy as np

import jax
import jax.numpy as jnp
from jax import lax
from jax.experimental import pallas as pl
from jax.experimental.pallas import tpu as pltpu
from jax.experimental.pallas import tpu_sc as plsc

B, H, W, C = 2, 224, 224, 96
C2 = 2 * C
BC = B * C
HW = H * W

ROW_TILE = 8
NT = H // ROW_TILE
PIX = ROW_TILE * W
HPIX = PIX // 2
HALF = HW // 2

NW = 32
PLANES_PER_W = BC // NW
NCHUNKS = 28
CHUNK = HW // NCHUNKS
ROWS_PER_CHUNK = CHUNK // W
VECS_PER_ROW = W // 16


def _grid_consts():
    l = np.arange(PIX)
    even = (l % 2 == 0)
    g = np.zeros((2, PIX), np.float32)
    for half in range(2):
        g[half] = np.where(even, half * (HALF // W) + (l // 2) // W,
                           (l // 2) % W)
    m = np.broadcast_to(even.astype(np.float32), (2, PIX)).copy()
    return g, m


_G_BASE, _G_TMASK = _grid_consts()


def _conv_body(x_ref, w_ref, b_ref, g_ref, m_ref, co, xt):
    t = pl.program_id(1)
    r0 = t * ROW_TILE
    acc = jnp.broadcast_to(b_ref[0][:, None], (C2, PIX))
    for dy in range(3):
        rows = x_ref[0, pl.ds(r0 + dy, ROW_TILE), :, :]
        for dx in range(3):
            blk = rows[:, dx:dx + W, :].reshape(PIX, C)
            acc = acc + lax.dot_general(
                w_ref[dy, dx], blk, (((0,), (1,)), ((), ())),
                preferred_element_type=jnp.float32)
    g = g_ref[...] + (t * (ROW_TILE // 2)).astype(jnp.float32) * m_ref[...]
    coords = acc.reshape(C, 2, PIX) + g[None, :, :]
    coords = jnp.minimum(jnp.maximum(coords, 0.0), jnp.float32(H - 1))
    co[0, :, :] = coords.reshape(C2, PIX)
    xmid = x_ref[0, pl.ds(r0 + 1, ROW_TILE), pl.ds(1, W), :]
    xt[0, :, :] = xmid.reshape(PIX, C).T


def _conv_coords(x, W_conv, b_conv):
    x_pad = jnp.pad(x, ((0, 0), (1, 1), (1, 1), (0, 0)))
    return pl.pallas_call(
        _conv_body,
        grid=(B, NT),
        in_specs=[
            pl.BlockSpec((1, H + 2, W + 2, C), lambda bi, ti: (bi, 0, 0, 0)),
            pl.BlockSpec((3, 3, C, C2), lambda bi, ti: (0, 0, 0, 0)),
            pl.BlockSpec((1, C2), lambda bi, ti: (0, 0)),
            pl.BlockSpec((2, PIX), lambda bi, ti: (0, 0)),
            pl.BlockSpec((2, PIX), lambda bi, ti: (0, 0)),
        ],
        out_specs=[
            pl.BlockSpec((1, C2, PIX), lambda bi, ti: (bi, 0, ti)),
            pl.BlockSpec((1, C, PIX), lambda bi, ti: (bi, 0, ti)),
        ],
        out_shape=[
            jax.ShapeDtypeStruct((B, C2, HW), jnp.float32),
            jax.ShapeDtypeStruct((B, C, HW), jnp.float32),
        ],
        compiler_params=pltpu.CompilerParams(
            vmem_limit_bytes=100 * 1024 * 1024),
    )(x_pad, W_conv, b_conv.reshape(1, C2), _G_BASE, _G_TMASK)


def _transpose_body(i_ref, o_ref):
    o_ref[0] = i_ref[0].T.reshape(ROW_TILE, W, C)


def _to_nhwc(planes):
    return pl.pallas_call(
        _transpose_body,
        grid=(B, NT),
        in_specs=[pl.BlockSpec((1, C, PIX), lambda bi, ti: (bi, 0, ti))],
        out_specs=pl.BlockSpec((1, ROW_TILE, W, C),
                               lambda bi, ti: (bi, ti, 0, 0)),
        out_shape=jax.ShapeDtypeStruct((B, H, W, C), jnp.float32),
    )(planes)


@functools.lru_cache(maxsize=1)
def _build_sc_sample():
    mesh = plsc.VectorSubcoreMesh(core_axis_name="c", subcore_axis_name="s")
    return functools.partial(
        pl.kernel,
        mesh=mesh,
        out_type=jax.ShapeDtypeStruct((BC, HW), jnp.float32),
        compiler_params=pltpu.CompilerParams(needs_layout_passes=False),
        scratch_types=[
            pltpu.VMEM((2, HW), jnp.float32),
            pltpu.VMEM((2, 2 * CHUNK), jnp.float32),
            pltpu.VMEM((2, CHUNK), jnp.float32),
            pltpu.SemaphoreType.DMA,
            pltpu.SemaphoreType.DMA,
            pltpu.SemaphoreType.DMA,
            pltpu.SemaphoreType.DMA,
            pltpu.SemaphoreType.DMA,
        ],
    )(_sc_sample_body)


def _sc_sample_body(xp_hbm, co_hbm, out_hbm, plane_v, c_v, res_v,
                    sem_p, sem_c0, sem_c1, sem_r0, sem_r1):
    wid = lax.axis_index("s") * 2 + lax.axis_index("c")
    iota = lax.iota(jnp.int32, 16)
    ev = iota * 2
    p0 = wid * PLANES_PER_W
    sem_c = (sem_c0, sem_c1)
    sem_r = (sem_r0, sem_r1)
    NG = PLANES_PER_W * NCHUNKS

    def coord_src(g):
        jj = g // NCHUNKS
        k = g - jj * NCHUNKS
        pn = p0 + jj
        bn = pn // C
        cin = pn - bn * C
        half = k // (NCHUNKS // 2)
        col0 = (k - half * (NCHUNKS // 2)) * (2 * CHUNK)
        return co_hbm.at[bn, 2 * cin + half, pl.ds(col0, 2 * CHUNK)]

    def out_dst(g):
        jj = g // NCHUNKS
        k = g - jj * NCHUNKS
        return out_hbm.at[p0 + jj, pl.ds(k * CHUNK, CHUNK)]

    pltpu.async_copy(xp_hbm.at[p0], plane_v.at[0], sem_p)
    pltpu.async_copy(coord_src(0), c_v.at[0], sem_c[0])

    def plane_body(j, _):
        p = p0 + j
        pp = j % 2
        ppv = jnp.full((16,), pp, jnp.int32)
        pltpu.make_async_copy(xp_hbm.at[p], plane_v.at[pp], sem_p).wait()

        @pl.when(j < PLANES_PER_W - 1)
        def _():
            pltpu.async_copy(
                xp_hbm.at[p + 1], plane_v.at[(j + 1) % 2], sem_p)

        def pair_body(m, _):
            for s in range(2):
                k = 2 * m + s
                g = j * NCHUNKS + k

                @pl.when(g + 1 < NG)
                def _():
                    pltpu.async_copy(
                        coord_src(g + 1), c_v.at[1 - s], sem_c[1 - s])

                pltpu.make_async_copy(
                    coord_src(g), c_v.at[s], sem_c[s]).wait()

                @pl.when(g >= 2)
                def _():
                    pltpu.make_async_copy(
                        res_v.at[s], out_dst(g - 2), sem_r[s]).wait()

                sv = jnp.full((16,), s, jnp.int32)

                def row_body(r, _):
                    for v in range(VECS_PER_ROW):
                        q = r * W + v * 16
                        cidx = ev + (2 * q)
                        cy = plsc.load_gather(c_v, [sv, cidx])
                        cx = plsc.load_gather(c_v, [sv, cidx + 1])
                        iy0 = cy.astype(jnp.int32)
                        ix0 = cx.astype(jnp.int32)
                        wy = cy - iy0.astype(jnp.float32)
                        wx = cx - ix0.astype(jnp.float32)
                        ady = jnp.where(iy0 < H - 1, W, 0)
                        adx = jnp.where(ix0 < W - 1, 1, 0)
                        i_lt = iy0 * W + ix0
                        i_rt = i_lt + ady
                        v_lt = plsc.load_gather(plane_v, [ppv, i_lt])
                        v_rt = plsc.load_gather(plane_v, [ppv, i_rt])
                        v_lb = plsc.load_gather(plane_v, [ppv, i_lt + adx])
                        v_rb = plsc.load_gather(plane_v, [ppv, i_rt + adx])
                        vt = v_lt + (v_rt - v_lt) * wy
                        vb = v_lb + (v_rb - v_lb) * wy
                        res_v[s, pl.ds(q, 16)] = vt + (vb - vt) * wx
                    return 0

                lax.fori_loop(0, ROWS_PER_CHUNK, row_body, 0, unroll=2)
                pltpu.async_copy(res_v.at[s], out_dst(g), sem_r[s])
            return 0

        lax.fori_loop(0, NCHUNKS // 2, pair_body, 0)
        return 0

    lax.fori_loop(0, PLANES_PER_W, plane_body, 0)
    pltpu.make_async_copy(res_v.at[0], out_dst(NG - 2), sem_r[0]).wait()
    pltpu.make_async_copy(res_v.at[1], out_dst(NG - 1), sem_r[1]).wait()


def kernel(x, W_conv, b_conv):
    co, xt = _conv_coords(x, W_conv, b_conv)
    x_bc = xt.reshape(BC, HW)
    planes = _build_sc_sample()(x_bc, co)
    return _to_nhwc(planes.reshape(B, C, HW))


# speedup vs baseline: 1.2138x; 1.1557x over previous
import functools

import numpy as np

import jax
import jax.numpy as jnp
from jax import lax
from jax.experimental import pallas as pl
from jax.experimental.pallas import tpu as pltpu
from jax.experimental.pallas import tpu_sc as plsc

B, H, W, C = 2, 224, 224, 96
C2 = 2 * C
BC = B * C
HW = H * W

ROW_TILE = 8
NT = H // ROW_TILE
PIX = ROW_TILE * W
HPIX = PIX // 2
HALF = HW // 2

NW = 32
PLANES_PER_W = BC // NW
NCHUNKS = 14
CHUNK = HW // NCHUNKS
ROWS_PER_CHUNK = CHUNK // W
VECS_PER_ROW = W // 16


def _grid_consts():
    l = np.arange(PIX)
    even = (l % 2 == 0)
    g = np.zeros((2, PIX), np.float32)
    for half in range(2):
        g[half] = np.where(even, half * (HALF // W) + (l // 2) // W,
                           (l // 2) % W)
    m = np.broadcast_to(even.astype(np.float32), (2, PIX)).copy()
    return g, m


_G_BASE, _G_TMASK = _grid_consts()


def _conv_body(x_ref, w_ref, b_ref, g_ref, m_ref, co, xt):
    t = pl.program_id(1)
    r0 = t * ROW_TILE
    acc = jnp.broadcast_to(b_ref[0][:, None], (C2, PIX))
    for dy in range(3):
        rows = x_ref[0, pl.ds(r0 + dy, ROW_TILE), :, :]
        for dx in range(3):
            blk = rows[:, dx:dx + W, :].reshape(PIX, C)
            acc = acc + lax.dot_general(
                w_ref[dy, dx], blk, (((0,), (1,)), ((), ())),
                preferred_element_type=jnp.float32)
    g = g_ref[...] + (t * (ROW_TILE // 2)).astype(jnp.float32) * m_ref[...]
    coords = acc.reshape(C, 2, PIX) + g[None, :, :]
    coords = jnp.minimum(jnp.maximum(coords, 0.0), jnp.float32(H - 1))
    co[0, :, :] = coords.reshape(C2, PIX)
    xmid = x_ref[0, pl.ds(r0 + 1, ROW_TILE), pl.ds(1, W), :]
    xt[0, :, :] = xmid.reshape(PIX, C).T


def _conv_coords(x, W_conv, b_conv):
    x_pad = jnp.pad(x, ((0, 0), (1, 1), (1, 1), (0, 0)))
    return pl.pallas_call(
        _conv_body,
        grid=(B, NT),
        in_specs=[
            pl.BlockSpec((1, H + 2, W + 2, C), lambda bi, ti: (bi, 0, 0, 0)),
            pl.BlockSpec((3, 3, C, C2), lambda bi, ti: (0, 0, 0, 0)),
            pl.BlockSpec((1, C2), lambda bi, ti: (0, 0)),
            pl.BlockSpec((2, PIX), lambda bi, ti: (0, 0)),
            pl.BlockSpec((2, PIX), lambda bi, ti: (0, 0)),
        ],
        out_specs=[
            pl.BlockSpec((1, C2, PIX), lambda bi, ti: (bi, 0, ti)),
            pl.BlockSpec((1, C, PIX), lambda bi, ti: (bi, 0, ti)),
        ],
        out_shape=[
            jax.ShapeDtypeStruct((B, C2, HW), jnp.float32),
            jax.ShapeDtypeStruct((B, C, HW), jnp.float32),
        ],
        compiler_params=pltpu.CompilerParams(
            vmem_limit_bytes=100 * 1024 * 1024),
    )(x_pad, W_conv, b_conv.reshape(1, C2), _G_BASE, _G_TMASK)


def _transpose_body(i_ref, o_ref):
    o_ref[0] = i_ref[0].T.reshape(ROW_TILE, W, C)


def _to_nhwc(planes):
    return pl.pallas_call(
        _transpose_body,
        grid=(B, NT),
        in_specs=[pl.BlockSpec((1, C, PIX), lambda bi, ti: (bi, 0, ti))],
        out_specs=pl.BlockSpec((1, ROW_TILE, W, C),
                               lambda bi, ti: (bi, ti, 0, 0)),
        out_shape=jax.ShapeDtypeStruct((B, H, W, C), jnp.float32),
    )(planes)


@functools.lru_cache(maxsize=1)
def _build_sc_sample():
    mesh = plsc.VectorSubcoreMesh(core_axis_name="c", subcore_axis_name="s")
    return functools.partial(
        pl.kernel,
        mesh=mesh,
        out_type=jax.ShapeDtypeStruct((BC, HW), jnp.float32),
        compiler_params=pltpu.CompilerParams(needs_layout_passes=False),
        scratch_types=[
            pltpu.VMEM((HW,), jnp.float32),
            pltpu.VMEM((4 * CHUNK,), jnp.float32),
            pltpu.VMEM((2 * CHUNK,), jnp.float32),
            pltpu.SemaphoreType.DMA,
            pltpu.SemaphoreType.DMA,
            pltpu.SemaphoreType.DMA,
            pltpu.SemaphoreType.DMA,
        ],
    )(_sc_sample_body)


def _sc_sample_body(xp_hbm, co_hbm, out_hbm, plane_v, c_v, res_v,
                    sem_c0, sem_c1, sem_r0, sem_r1):
    wid = lax.axis_index("s") * 2 + lax.axis_index("c")
    iota = lax.iota(jnp.int32, 16)
    ev = iota * 2
    p0 = wid * PLANES_PER_W
    sem_c = (sem_c0, sem_c1)
    sem_r = (sem_r0, sem_r1)
    NG = PLANES_PER_W * NCHUNKS

    def coord_src(g):
        jj = g // NCHUNKS
        k = g - jj * NCHUNKS
        pn = p0 + jj
        bn = pn // C
        cin = pn - bn * C
        half = k // (NCHUNKS // 2)
        col0 = (k - half * (NCHUNKS // 2)) * (2 * CHUNK)
        return co_hbm.at[bn, 2 * cin + half, pl.ds(col0, 2 * CHUNK)]

    def out_dst(g):
        jj = g // NCHUNKS
        k = g - jj * NCHUNKS
        return out_hbm.at[p0 + jj, pl.ds(k * CHUNK, CHUNK)]

    def c_slot(s):
        return c_v.at[pl.ds(s * 2 * CHUNK, 2 * CHUNK)]

    def r_slot(s):
        return res_v.at[pl.ds(s * CHUNK, CHUNK)]

    pltpu.async_copy(coord_src(0), c_slot(0), sem_c[0])

    def plane_body(j, _):
        p = p0 + j
        pltpu.sync_copy(xp_hbm.at[p], plane_v)

        def pair_body(m, _):
            for s in range(2):
                k = 2 * m + s
                g = j * NCHUNKS + k

                @pl.when(g + 1 < NG)
                def _():
                    pltpu.async_copy(
                        coord_src(g + 1), c_slot(1 - s), sem_c[1 - s])

                pltpu.make_async_copy(
                    coord_src(g), c_slot(s), sem_c[s]).wait()

                @pl.when(g >= 2)
                def _():
                    pltpu.make_async_copy(
                        r_slot(s), out_dst(g - 2), sem_r[s]).wait()

                cbase = s * 2 * CHUNK
                rbase = s * CHUNK

                def row_body(r, _):
                    for v in range(VECS_PER_ROW):
                        q = r * W + v * 16
                        cidx = ev + (cbase + 2 * q)
                        cy = plsc.load_gather(c_v, [cidx])
                        cx = plsc.load_gather(c_v, [cidx + 1])
                        iy0 = cy.astype(jnp.int32)
                        ix0 = cx.astype(jnp.int32)
                        wy = cy - iy0.astype(jnp.float32)
                        wx = cx - ix0.astype(jnp.float32)
                        ady = jnp.where(iy0 < H - 1, W, 0)
                        adx = jnp.where(ix0 < W - 1, 1, 0)
                        i_lt = iy0 * W + ix0
                        i_rt = i_lt + ady
                        v_lt = plsc.load_gather(plane_v, [i_lt])
                        v_rt = plsc.load_gather(plane_v, [i_rt])
                        v_lb = plsc.load_gather(plane_v, [i_lt + adx])
                        v_rb = plsc.load_gather(plane_v, [i_rt + adx])
                        vt = v_lt + (v_rt - v_lt) * wy
                        vb = v_lb + (v_rb - v_lb) * wy
                        res_v[pl.ds(rbase + q, 16)] = vt + (vb - vt) * wx
                    return 0

                lax.fori_loop(0, ROWS_PER_CHUNK, row_body, 0, unroll=2)
                pltpu.async_copy(r_slot(s), out_dst(g), sem_r[s])
            return 0

        lax.fori_loop(0, NCHUNKS // 2, pair_body, 0)
        return 0

    lax.fori_loop(0, PLANES_PER_W, plane_body, 0)
    pltpu.make_async_copy(r_slot(0), out_dst(NG - 2), sem_r[0]).wait()
    pltpu.make_async_copy(r_slot(1), out_dst(NG - 1), sem_r[1]).wait()


def kernel(x, W_conv, b_conv):
    co, xt = _conv_coords(x, W_conv, b_conv)
    x_bc = xt.reshape(BC, HW)
    planes = _build_sc_sample()(x_bc, co)
    return _to_nhwc(planes.reshape(B, C, HW))


# speedup vs baseline: 1.3587x; 1.1194x over previous
import functools

import numpy as np

import jax
import jax.numpy as jnp
from jax import lax
from jax.experimental import pallas as pl
from jax.experimental.pallas import tpu as pltpu
from jax.experimental.pallas import tpu_sc as plsc

B, H, W, C = 2, 224, 224, 96
C2 = 2 * C
BC = B * C
HW = H * W

ROW_TILE = 8
NT = H // ROW_TILE
PIX = ROW_TILE * W
HPIX = PIX // 2
HALF = HW // 2

NW = 32
PLANES_PER_W = BC // NW
NCHUNKS = 14
CHUNK = HW // NCHUNKS
ROWS_PER_CHUNK = CHUNK // W
VECS_PER_ROW = W // 16


def _grid_consts():
    l = np.arange(PIX)
    even = (l % 2 == 0)
    g = np.zeros((2, PIX), np.float32)
    for half in range(2):
        g[half] = np.where(even, half * (HALF // W) + (l // 2) // W,
                           (l // 2) % W)
    m = np.broadcast_to(even.astype(np.float32), (2, PIX)).copy()
    return g, m


_G_BASE, _G_TMASK = _grid_consts()


def _conv_body(x_ref, w_ref, b_ref, g_ref, m_ref, co, xt):
    t = pl.program_id(1)
    r0 = t * ROW_TILE
    acc = jnp.broadcast_to(b_ref[0][:, None], (C2, PIX))
    for dy in range(3):
        rows = x_ref[0, pl.ds(r0 + dy, ROW_TILE), :, :]
        for dx in range(3):
            blk = rows[:, dx:dx + W, :].reshape(PIX, C)
            acc = acc + lax.dot_general(
                w_ref[dy, dx], blk, (((0,), (1,)), ((), ())),
                preferred_element_type=jnp.float32)
    g = g_ref[...] + (t * (ROW_TILE // 2)).astype(jnp.float32) * m_ref[...]
    coords = acc.reshape(C, 2, PIX) + g[None, :, :]
    coords = jnp.minimum(jnp.maximum(coords, 0.0), jnp.float32(H - 1))
    co[0, :, :] = coords.reshape(C2, PIX)
    xmid = x_ref[0, pl.ds(r0 + 1, ROW_TILE), pl.ds(1, W), :]
    xt[0, :, :] = xmid.reshape(PIX, C).T


def _conv_coords_one(x_pad_b, W_conv, b2, g, m):
    return pl.pallas_call(
        _conv_body,
        grid=(1, NT),
        in_specs=[
            pl.BlockSpec((1, H + 2, W + 2, C), lambda bi, ti: (bi, 0, 0, 0)),
            pl.BlockSpec((3, 3, C, C2), lambda bi, ti: (0, 0, 0, 0)),
            pl.BlockSpec((1, C2), lambda bi, ti: (0, 0)),
            pl.BlockSpec((2, PIX), lambda bi, ti: (0, 0)),
            pl.BlockSpec((2, PIX), lambda bi, ti: (0, 0)),
        ],
        out_specs=[
            pl.BlockSpec((1, C2, PIX), lambda bi, ti: (bi, 0, ti)),
            pl.BlockSpec((1, C, PIX), lambda bi, ti: (bi, 0, ti)),
        ],
        out_shape=[
            jax.ShapeDtypeStruct((1, C2, HW), jnp.float32),
            jax.ShapeDtypeStruct((1, C, HW), jnp.float32),
        ],
        compiler_params=pltpu.CompilerParams(
            vmem_limit_bytes=100 * 1024 * 1024),
    )(x_pad_b, W_conv, b2, g, m)


def _conv_coords(x, W_conv, b_conv):
    b2 = b_conv.reshape(1, C2)
    cos, xts = [], []
    for bi in range(B):
        x_pad_b = jnp.pad(x[bi:bi + 1], ((0, 0), (1, 1), (1, 1), (0, 0)))
        co_b, xt_b = _conv_coords_one(x_pad_b, W_conv, b2, _G_BASE, _G_TMASK)
        cos.append(co_b.reshape(C2, HW))
        xts.append(xt_b.reshape(C, HW))
    return cos, xts


def _transpose_body(i_ref, o_ref):
    o_ref[0] = i_ref[0].T.reshape(ROW_TILE, W, C)


def _to_nhwc(planes):
    return pl.pallas_call(
        _transpose_body,
        grid=(B, NT),
        in_specs=[pl.BlockSpec((1, C, PIX), lambda bi, ti: (bi, 0, ti))],
        out_specs=pl.BlockSpec((1, ROW_TILE, W, C),
                               lambda bi, ti: (bi, ti, 0, 0)),
        out_shape=jax.ShapeDtypeStruct((B, H, W, C), jnp.float32),
    )(planes)


@functools.lru_cache(maxsize=1)
def _build_sc_sample():
    mesh = plsc.VectorSubcoreMesh(core_axis_name="c", subcore_axis_name="s")
    return functools.partial(
        pl.kernel,
        mesh=mesh,
        out_type=jax.ShapeDtypeStruct((BC, HW), jnp.float32),
        compiler_params=pltpu.CompilerParams(needs_layout_passes=False),
        scratch_types=[
            pltpu.VMEM((HW,), jnp.float32),
            pltpu.VMEM((4 * CHUNK,), jnp.float32),
            pltpu.VMEM((2 * CHUNK,), jnp.float32),
            pltpu.SemaphoreType.DMA,
            pltpu.SemaphoreType.DMA,
            pltpu.SemaphoreType.DMA,
            pltpu.SemaphoreType.DMA,
        ],
    )(_sc_sample_body)


def _sc_sample_body(xt0, xt1, co0, co1, out_hbm, plane_v, c_v, res_v,
                    sem_c0, sem_c1, sem_r0, sem_r1):
    wid = lax.axis_index("s") * 2 + lax.axis_index("c")
    iota = lax.iota(jnp.int32, 16)
    ev = iota * 2
    sem_c = (sem_c0, sem_c1)
    sem_r = (sem_r0, sem_r1)
    NG = PLANES_PER_W * NCHUNKS
    WPB = (NW // B)

    def run(xp_ref, co_ref, bb):
        ci0 = (wid - bb * WPB) * PLANES_PER_W

        def coord_src(g):
            jj = g // NCHUNKS
            k = g - jj * NCHUNKS
            cin = ci0 + jj
            half = k // (NCHUNKS // 2)
            col0 = (k - half * (NCHUNKS // 2)) * (2 * CHUNK)
            return co_ref.at[2 * cin + half, pl.ds(col0, 2 * CHUNK)]

        def out_dst(g):
            jj = g // NCHUNKS
            k = g - jj * NCHUNKS
            return out_hbm.at[bb * C + ci0 + jj, pl.ds(k * CHUNK, CHUNK)]

        def c_slot(s):
            return c_v.at[pl.ds(s * 2 * CHUNK, 2 * CHUNK)]

        def r_slot(s):
            return res_v.at[pl.ds(s * CHUNK, CHUNK)]

        pltpu.async_copy(coord_src(0), c_slot(0), sem_c[0])

        def plane_body(j, _):
            pltpu.sync_copy(xp_ref.at[ci0 + j], plane_v)

            def pair_body(m, _):
                for s in range(2):
                    k = 2 * m + s
                    g = j * NCHUNKS + k

                    @pl.when(g + 1 < NG)
                    def _():
                        pltpu.async_copy(
                            coord_src(g + 1), c_slot(1 - s), sem_c[1 - s])

                    pltpu.make_async_copy(
                        coord_src(g), c_slot(s), sem_c[s]).wait()

                    @pl.when(g >= 2)
                    def _():
                        pltpu.make_async_copy(
                            r_slot(s), out_dst(g - 2), sem_r[s]).wait()

                    cbase = s * 2 * CHUNK
                    rbase = s * CHUNK

                    def row_body(r, _):
                        for v in range(VECS_PER_ROW):
                            q = r * W + v * 16
                            cidx = ev + (cbase + 2 * q)
                            cy = plsc.load_gather(c_v, [cidx])
                            cx = plsc.load_gather(c_v, [cidx + 1])
                            iy0 = cy.astype(jnp.int32)
                            ix0 = cx.astype(jnp.int32)
                            wy = cy - iy0.astype(jnp.float32)
                            wx = cx - ix0.astype(jnp.float32)
                            ady = jnp.where(iy0 < H - 1, W, 0)
                            adx = jnp.where(ix0 < W - 1, 1, 0)
                            i_lt = iy0 * W + ix0
                            i_rt = i_lt + ady
                            v_lt = plsc.load_gather(plane_v, [i_lt])
                            v_rt = plsc.load_gather(plane_v, [i_rt])
                            v_lb = plsc.load_gather(plane_v, [i_lt + adx])
                            v_rb = plsc.load_gather(plane_v, [i_rt + adx])
                            vt = v_lt + (v_rt - v_lt) * wy
                            vb = v_lb + (v_rb - v_lb) * wy
                            res_v[pl.ds(rbase + q, 16)] = vt + (vb - vt) * wx
                        return 0

                    lax.fori_loop(0, ROWS_PER_CHUNK, row_body, 0, unroll=2)
                    pltpu.async_copy(r_slot(s), out_dst(g), sem_r[s])
                return 0

            lax.fori_loop(0, NCHUNKS // 2, pair_body, 0)
            return 0

        lax.fori_loop(0, PLANES_PER_W, plane_body, 0)
        pltpu.make_async_copy(r_slot(0), out_dst(NG - 2), sem_r[0]).wait()
        pltpu.make_async_copy(r_slot(1), out_dst(NG - 1), sem_r[1]).wait()

    @pl.when(wid < WPB)
    def _():
        run(xt0, co0, 0)

    @pl.when(wid >= WPB)
    def _():
        run(xt1, co1, 1)


def kernel(x, W_conv, b_conv):
    cos, xts = _conv_coords(x, W_conv, b_conv)
    planes = _build_sc_sample()(xts[0], xts[1], cos[0], cos[1])
    return _to_nhwc(planes.reshape(B, C, HW))
